# baseline (speedup 1.0000x reference)
import functools
import jax
import jax.numpy as jnp
from jax import lax
from jax.experimental import pallas as pl
from jax.experimental.pallas import tpu as pltpu
from jax.experimental.pallas import tpu_sc as plsc

_B, _L, _D, _A, _ACT = 16, 1024, 256, 128, 1024
_H1, _H2 = 512, 1024

_SC_ROWS = 8
_WPR = 32 // _SC_ROWS
_SEG = _L // _WPR
_CL = 128
_NCH = _SEG // _CL
_NV = _D // 16

_TC_ROWS = _B - _SC_ROWS
_R2 = 4
_TC_STEPS = _TC_ROWS // _R2


def _prep_body(now_ref, wa_ref, ba_ref, ww_ref, bw_ref,
               v_out, c16_out, cslot_out):
    wa = wa_ref[...]
    ww_top = ww_ref[0:_A, :]
    ww_bot = ww_ref[_A:2 * _A, :]
    v_out[...] = lax.dot_general(ww_bot, wa, (((0,), (1,)), ((), ())),
                                 preferred_element_type=jnp.float32)
    u = jnp.dot(wa, ww_top, preferred_element_type=jnp.float32)
    const = jnp.dot(ba_ref[...], ww_top, preferred_element_type=jnp.float32)
    const = const + jnp.dot(ba_ref[...], ww_bot, preferred_element_type=jnp.float32)
    const = const + bw_ref[...]
    cT = lax.dot_general(u, now_ref[...], (((0,), (1,)), ((), ())),
                         preferred_element_type=jnp.float32)
    c16_out[...] = cT + const
    c_col = jnp.dot(now_ref[...], u, preferred_element_type=jnp.float32) + const
    for s in range(_TC_STEPS):
        lo = _SC_ROWS + s * _R2
        cslot_out[pl.ds(s * 8, _R2), :] = c_col[lo:lo + _R2, :]


def _prep_call(now, wa, ba2, ww, bw2):
    const = lambda *_: (0, 0)
    return pl.pallas_call(
        _prep_body,
        grid=(1,),
        in_specs=[pl.BlockSpec((_B, _D), const),
                  pl.BlockSpec((_D, _A), const),
                  pl.BlockSpec((1, _A), const),
                  pl.BlockSpec((2 * _A, 1), const),
                  pl.BlockSpec((1, 1), const)],
        out_specs=[pl.BlockSpec((1, _D), const),
                   pl.BlockSpec((1, _B), const),
                   pl.BlockSpec((_TC_STEPS * 8, 1), const)],
        out_shape=[jax.ShapeDtypeStruct((1, _D), jnp.float32),
                   jax.ShapeDtypeStruct((1, _B), jnp.float32),
                   jax.ShapeDtypeStruct((_TC_STEPS * 8, 1), jnp.float32)],
    )(now, wa, ba2, ww, bw2)


_GDN = lax.GatherDimensionNumbers(offset_dims=(), collapsed_slice_dims=(0,),
                                  start_index_map=(0,))


def _lperm(t, idxv):
    return lax.gather(t, idxv[:, None], _GDN, (1,),
                      mode=lax.GatherScatterMode.PROMISE_IN_BOUNDS)


def _lanesum(t):
    ii = lax.iota(jnp.int32, 16)
    for sh in (8, 4, 2, 1):
        t = t + _lperm(t, jnp.bitwise_xor(ii, sh))
    return t


def _lanemax(t):
    ii = lax.iota(jnp.int32, 16)
    for sh in (8, 4, 2, 1):
        t = jnp.maximum(t, _lperm(t, jnp.bitwise_xor(ii, sh)))
    return t


def _sc_attn_body(ne_hbm, v_hbm, c_hbm, aggp_hbm, stats_hbm,
                  buf0, buf1, v_vmem, c_vmem, s_buf, out_vmem, st_vmem,
                  sem0, sem1):
    w = lax.axis_index("c") * 16 + lax.axis_index("s")
    row = w // _WPR
    seg = w % _WPR
    base = row * _L + seg * _SEG

    pltpu.sync_copy(v_hbm, v_vmem)
    pltpu.sync_copy(c_hbm, c_vmem)
    idx = lax.iota(jnp.int32, 16)
    cv = c_vmem[pl.ds(0, _B)]
    row16 = jnp.full((_B,), row, jnp.int32)
    c16 = _lanesum(jnp.where(lax.iota(jnp.int32, _B) == row16, cv,
                             jnp.zeros((_B,), jnp.float32)))
    vvecs = [v_vmem[pl.ds(16 * i, 16)] for i in range(_NV)]
    xp = {s: jnp.bitwise_xor(idx, s) for s in (1, 2, 4, 8)}
    msk = {s: jnp.bitwise_and(idx, s) == 0 for s in (1, 2, 4, 8)}
    jbc = [jnp.full((16,), j, jnp.int32) for j in range(16)]

    bufs = [buf0, buf1]
    sems = [sem0, sem1]
    copies = [None, None]
    copies[0] = pltpu.async_copy(ne_hbm.at[pl.ds(base, _CL)], buf0, sem0)

    m_run = jnp.full((16,), -1e30, jnp.float32)
    esum = jnp.zeros((16,), jnp.float32)
    acc = [jnp.zeros((16,), jnp.float32) for _ in range(_NV)]

    for ch in range(_NCH):
        buf = bufs[ch % 2]
        copies[ch % 2].wait()
        if ch + 1 < _NCH:
            copies[(ch + 1) % 2] = pltpu.async_copy(
                ne_hbm.at[pl.ds(base + (ch + 1) * _CL, _CL)],
                bufs[(ch + 1) % 2], sems[(ch + 1) % 2])

        def score_body(g, _, buf=buf):
            ts = []
            for j in range(16):
                l = g * 16 + j
                t = buf[l, pl.ds(0, 16)] * vvecs[0]
                for i in range(1, _NV):
                    t = t + buf[l, pl.ds(16 * i, 16)] * vvecs[i]
                ts.append(t)
            for s in (1, 2, 4, 8):
                nxt = []
                for k in range(0, len(ts), 2):
                    a, b = ts[k], ts[k + 1]
                    nxt.append(jnp.where(msk[s],
                                         a + _lperm(a, xp[s]),
                                         b + _lperm(b, xp[s])))
                ts = nxt
            sv = ts[0] + c16
            sv = jnp.where(sv >= 0.0, sv, 0.2 * sv)
            s_buf[pl.ds(g * 16, 16)] = sv
            return 0
        lax.fori_loop(0, _CL // 16, score_body, 0)

        mv = s_buf[pl.ds(0, 16)]
        for j in range(1, _CL // 16):
            mv = jnp.maximum(mv, s_buf[pl.ds(16 * j, 16)])
        m_new = jnp.maximum(m_run, _lanemax(mv))
        scale = jnp.exp(m_run - m_new)
        esum = esum * scale
        for j in range(_CL // 16):
            e = jnp.exp(s_buf[pl.ds(16 * j, 16)] - m_new)
            s_buf[pl.ds(16 * j, 16)] = e
            esum = esum + e
        acc = [a * scale for a in acc]

        def weight_body(g, accs, buf=buf):
            e_vec = s_buf[pl.ds(g * 16, 16)]
            for j in range(16):
                l = g * 16 + j
                e16 = _lperm(e_vec, jbc[j])
                accs = [accs[i] + buf[l, pl.ds(16 * i, 16)] * e16
                        for i in range(_NV)]
            return accs
        acc = lax.fori_loop(0, _CL // 16, weight_body, acc)
        m_run = m_new

    for i in range(_NV):
        out_vmem[0, pl.ds(16 * i, 16)] = acc[i]
    pltpu.sync_copy(out_vmem, aggp_hbm.at[pl.ds(w, 1)])

    stats = jnp.where(idx == 0, m_run,
                      jnp.where(idx == 1, _lanesum(esum),
                                jnp.zeros((16,), jnp.float32)))
    st_vmem[0, :] = stats
    pltpu.sync_copy(st_vmem, stats_hbm.at[pl.ds(w, 1)])


_sc_attn_call = functools.partial(
    pl.kernel,
    out_type=[jax.ShapeDtypeStruct((32, _D), jnp.float32),
              jax.ShapeDtypeStruct((32, 16), jnp.float32)],
    mesh=plsc.VectorSubcoreMesh(core_axis_name="c", subcore_axis_name="s"),
    scratch_types=[pltpu.VMEM((_CL, _D), jnp.float32),
                   pltpu.VMEM((_CL, _D), jnp.float32),
                   pltpu.VMEM((_D,), jnp.float32),
                   pltpu.VMEM((_B,), jnp.float32),
                   pltpu.VMEM((_CL,), jnp.float32),
                   pltpu.VMEM((1, _D), jnp.float32),
                   pltpu.VMEM((1, 16), jnp.float32),
                   pltpu.SemaphoreType.DMA,
                   pltpu.SemaphoreType.DMA],
)(_sc_attn_body)


def _tc_attn_body(ne_ref, v_ref, c_ref, agg_ref):
    b = pl.program_id(0)
    ne = ne_ref[...]
    scores = lax.dot_general(v_ref[...], ne, (((1,), (1,)), ((), ())),
                             preferred_element_type=jnp.float32)
    c_blk = c_ref[pl.ds(pl.multiple_of(b * 8, 8), _R2), :]
    aggs = []
    for r in range(_R2):
        s = scores[:, r * _L:(r + 1) * _L] + c_blk[r:r + 1, :]
        s = jnp.where(s >= 0.0, s, 0.2 * s)
        m = jnp.max(s, axis=1, keepdims=True)
        e = jnp.exp(s - m)
        alpha = e / jnp.sum(e, axis=1, keepdims=True)
        aggs.append(jnp.dot(alpha, ne[r * _L:(r + 1) * _L, :],
                            preferred_element_type=jnp.float32))
    agg_ref[0] = jnp.concatenate(aggs, axis=0)


def _tc_attn_call(ne_flat, v2, cslots):
    const = lambda *_: (0, 0)
    out3 = pl.pallas_call(
        _tc_attn_body,
        grid=(_TC_STEPS,),
        in_specs=[pl.BlockSpec((_R2 * _L, _D), lambda b: (b + _SC_ROWS // _R2, 0)),
                  pl.BlockSpec((1, _D), const),
                  pl.BlockSpec((_TC_STEPS * 8, 1), const)],
        out_specs=pl.BlockSpec((1, _R2, _D), lambda b: (b, 0, 0)),
        out_shape=jax.ShapeDtypeStruct((_TC_STEPS, _R2, _D), jnp.float32),
    )(ne_flat, v2, cslots)
    return out3.reshape(_TC_ROWS, _D)


def _mlp_body(state_ref, aggtc_ref, aggp_ref, stats_ref,
              w1_ref, b1_ref, w2_ref, b2_ref, w3_ref, b3_ref, out_ref):
    rows = []
    for r in range(_SC_ROWS):
        sl = aggp_ref[pl.ds(r * _WPR, _WPR), :]
        st = stats_ref[pl.ds(r * _WPR, _WPR), :]
        m4 = st[:, 0:1]
        s4 = st[:, 1:2]
        m = jnp.max(m4, axis=0, keepdims=True)
        scale = jnp.exp(m4 - m)
        num = jnp.sum(scale * sl, axis=0, keepdims=True)
        den = jnp.sum(scale * s4, axis=0, keepdims=True)
        rows.append(num / den)
    ag = jnp.concatenate(rows + [aggtc_ref[...]], axis=0)

    st_full = state_ref[...]
    h = jnp.dot(st_full, w1_ref[0:2 * _D, :], preferred_element_type=jnp.float32)
    h = h + jnp.dot(ag, w1_ref[2 * _D:3 * _D, :], preferred_element_type=jnp.float32)
    h = jax.nn.relu(h + b1_ref[...])
    h = jax.nn.relu(jnp.dot(h, w2_ref[...], preferred_element_type=jnp.float32) + b2_ref[...])
    logits = jnp.dot(h, w3_ref[...], preferred_element_type=jnp.float32) + b3_ref[...]
    z = logits - jnp.max(logits, axis=1, keepdims=True)
    ez = jnp.exp(z)
    out_ref[...] = ez / jnp.sum(ez, axis=1, keepdims=True)


def _mlp_call(state, agg_tc, aggp, stats, W1, b12, W2, b22, W3, b32):
    const = lambda *_: (0, 0)
    return pl.pallas_call(
        _mlp_body,
        grid=(1,),
        in_specs=[pl.BlockSpec((_B, 2 * _D), const),
                  pl.BlockSpec((_TC_ROWS, _D), const),
                  pl.BlockSpec((32, _D), const),
                  pl.BlockSpec((32, 16), const),
                  pl.BlockSpec((3 * _D, _H1), const),
                  pl.BlockSpec((1, _H1), const),
                  pl.BlockSpec((_H1, _H2), const),
                  pl.BlockSpec((1, _H2), const),
                  pl.BlockSpec((_H2, _ACT), const),
                  pl.BlockSpec((1, _ACT), const)],
        out_specs=pl.BlockSpec((_B, _ACT), const),
        out_shape=jax.ShapeDtypeStruct((_B, _ACT), jnp.float32),
    )(state, agg_tc, aggp, stats, W1, b12, W2, b22, W3, b32)


def kernel(state, now_embedding, neighbour_embeddings, Wa, ba, Ww, bw, W1, b1, W2, b2, W3, b3):
    ne_flat = neighbour_embeddings.reshape(_B * _L, _D)
    ba2 = ba.reshape(1, _A)
    bw2 = bw.reshape(1, 1)
    b12 = b1.reshape(1, _H1)
    b22 = b2.reshape(1, _H2)
    b32 = b3.reshape(1, _ACT)

    v2, c16, cslots = _prep_call(now_embedding, Wa, ba2, Ww, bw2)
    aggp, stats = _sc_attn_call(ne_flat, v2.reshape(_D), c16.reshape(_B))
    agg_tc = _tc_attn_call(ne_flat, v2, cslots)
    return _mlp_call(state, agg_tc, aggp, stats, W1, b12, W2, b22, W3, b32)


# speedup vs baseline: 1.2972x; 1.2972x over previous
import functools
import jax
import jax.numpy as jnp
from jax import lax
from jax.experimental import pallas as pl
from jax.experimental.pallas import tpu as pltpu
from jax.experimental.pallas import tpu_sc as plsc

_B, _L, _D, _A, _ACT = 16, 1024, 256, 128, 1024
_H1, _H2 = 512, 1024

_SC_ROWS = 4
_WPR = 32 // _SC_ROWS
_SEG = _L // _WPR
_CL = 128
_NCH = _SEG // _CL
_NV = _D // 16

_TC_ROWS = _B - _SC_ROWS
_R2 = 4
_TC_STEPS = _TC_ROWS // _R2


def _prep_body(now_ref, wa_ref, ba_ref, ww_ref, bw_ref,
               v_out, c16_out, cslot_out):
    wa = wa_ref[...]
    ww_top = ww_ref[0:_A, :]
    ww_bot = ww_ref[_A:2 * _A, :]
    v_out[...] = lax.dot_general(ww_bot, wa, (((0,), (1,)), ((), ())),
                                 preferred_element_type=jnp.float32)
    u = jnp.dot(wa, ww_top, preferred_element_type=jnp.float32)
    const = jnp.dot(ba_ref[...], ww_top, preferred_element_type=jnp.float32)
    const = const + jnp.dot(ba_ref[...], ww_bot, preferred_element_type=jnp.float32)
    const = const + bw_ref[...]
    cT = lax.dot_general(u, now_ref[...], (((0,), (1,)), ((), ())),
                         preferred_element_type=jnp.float32)
    c16_out[...] = cT + const
    c_col = jnp.dot(now_ref[...], u, preferred_element_type=jnp.float32) + const
    for s in range(_TC_STEPS):
        lo = _SC_ROWS + s * _R2
        cslot_out[pl.ds(s * 8, _R2), :] = c_col[lo:lo + _R2, :]


def _prep_call(now, wa, ba2, ww, bw2):
    const = lambda *_: (0, 0)
    return pl.pallas_call(
        _prep_body,
        grid=(1,),
        in_specs=[pl.BlockSpec((_B, _D), const),
                  pl.BlockSpec((_D, _A), const),
                  pl.BlockSpec((1, _A), const),
                  pl.BlockSpec((2 * _A, 1), const),
                  pl.BlockSpec((1, 1), const)],
        out_specs=[pl.BlockSpec((1, _D), const),
                   pl.BlockSpec((1, _B), const),
                   pl.BlockSpec((_TC_STEPS * 8, 1), const)],
        out_shape=[jax.ShapeDtypeStruct((1, _D), jnp.float32),
                   jax.ShapeDtypeStruct((1, _B), jnp.float32),
                   jax.ShapeDtypeStruct((_TC_STEPS * 8, 1), jnp.float32)],
    )(now, wa, ba2, ww, bw2)


_GDN = lax.GatherDimensionNumbers(offset_dims=(), collapsed_slice_dims=(0,),
                                  start_index_map=(0,))


def _lperm(t, idxv):
    return lax.gather(t, idxv[:, None], _GDN, (1,),
                      mode=lax.GatherScatterMode.PROMISE_IN_BOUNDS)


def _lanesum(t):
    ii = lax.iota(jnp.int32, 16)
    for sh in (8, 4, 2, 1):
        t = t + _lperm(t, jnp.bitwise_xor(ii, sh))
    return t


def _lanemax(t):
    ii = lax.iota(jnp.int32, 16)
    for sh in (8, 4, 2, 1):
        t = jnp.maximum(t, _lperm(t, jnp.bitwise_xor(ii, sh)))
    return t


def _sc_attn_body(ne_hbm, v_hbm, c_hbm, aggp_hbm, stats_hbm,
                  buf0, buf1, v_vmem, c_vmem, s_buf, out_vmem, st_vmem,
                  sem0, sem1):
    w = lax.axis_index("c") * 16 + lax.axis_index("s")
    row = w // _WPR
    seg = w % _WPR
    base = row * _L + seg * _SEG

    pltpu.sync_copy(v_hbm, v_vmem)
    pltpu.sync_copy(c_hbm, c_vmem)
    idx = lax.iota(jnp.int32, 16)
    cv = c_vmem[pl.ds(0, _B)]
    row16 = jnp.full((_B,), row, jnp.int32)
    c16 = _lanesum(jnp.where(lax.iota(jnp.int32, _B) == row16, cv,
                             jnp.zeros((_B,), jnp.float32)))
    vvecs = [v_vmem[pl.ds(16 * i, 16)] for i in range(_NV)]
    xp = {s: jnp.bitwise_xor(idx, s) for s in (1, 2, 4, 8)}
    msk = {s: jnp.bitwise_and(idx, s) == 0 for s in (1, 2, 4, 8)}
    jbc = [jnp.full((16,), j, jnp.int32) for j in range(16)]

    bufs = [buf0, buf1]
    sems = [sem0, sem1]
    copies = [None, None]
    copies[0] = pltpu.async_copy(ne_hbm.at[pl.ds(base, _CL)], buf0, sem0)

    m_run = jnp.full((16,), -1e30, jnp.float32)
    esum = jnp.zeros((16,), jnp.float32)
    acc = [jnp.zeros((16,), jnp.float32) for _ in range(_NV)]

    for ch in range(_NCH):
        buf = bufs[ch % 2]
        copies[ch % 2].wait()
        if ch + 1 < _NCH:
            copies[(ch + 1) % 2] = pltpu.async_copy(
                ne_hbm.at[pl.ds(base + (ch + 1) * _CL, _CL)],
                bufs[(ch + 1) % 2], sems[(ch + 1) % 2])

        def score_body(g, _, buf=buf):
            ts = []
            for j in range(16):
                l = g * 16 + j
                t = buf[l, pl.ds(0, 16)] * vvecs[0]
                for i in range(1, _NV):
                    t = t + buf[l, pl.ds(16 * i, 16)] * vvecs[i]
                ts.append(t)
            for s in (1, 2, 4, 8):
                nxt = []
                for k in range(0, len(ts), 2):
                    a, b = ts[k], ts[k + 1]
                    nxt.append(jnp.where(msk[s],
                                         a + _lperm(a, xp[s]),
                                         b + _lperm(b, xp[s])))
                ts = nxt
            sv = ts[0] + c16
            sv = jnp.where(sv >= 0.0, sv, 0.2 * sv)
            s_buf[pl.ds(g * 16, 16)] = sv
            return 0
        lax.fori_loop(0, _CL // 16, score_body, 0)

        mv = s_buf[pl.ds(0, 16)]
        for j in range(1, _CL // 16):
            mv = jnp.maximum(mv, s_buf[pl.ds(16 * j, 16)])
        m_new = jnp.maximum(m_run, _lanemax(mv))
        scale = jnp.exp(m_run - m_new)
        esum = esum * scale
        for j in range(_CL // 16):
            e = jnp.exp(s_buf[pl.ds(16 * j, 16)] - m_new)
            s_buf[pl.ds(16 * j, 16)] = e
            esum = esum + e
        acc = [a * scale for a in acc]

        def weight_body(g, accs, buf=buf):
            e_vec = s_buf[pl.ds(g * 16, 16)]
            for j in range(16):
                l = g * 16 + j
                e16 = _lperm(e_vec, jbc[j])
                accs = [accs[i] + buf[l, pl.ds(16 * i, 16)] * e16
                        for i in range(_NV)]
            return accs
        acc = lax.fori_loop(0, _CL // 16, weight_body, acc)
        m_run = m_new

    for i in range(_NV):
        out_vmem[0, pl.ds(16 * i, 16)] = acc[i]
    pltpu.sync_copy(out_vmem, aggp_hbm.at[pl.ds(w, 1)])

    stats = jnp.where(idx == 0, m_run,
                      jnp.where(idx == 1, _lanesum(esum),
                                jnp.zeros((16,), jnp.float32)))
    st_vmem[0, :] = stats
    pltpu.sync_copy(st_vmem, stats_hbm.at[pl.ds(w, 1)])


_sc_attn_call = functools.partial(
    pl.kernel,
    out_type=[jax.ShapeDtypeStruct((32, _D), jnp.float32),
              jax.ShapeDtypeStruct((32, 16), jnp.float32)],
    mesh=plsc.VectorSubcoreMesh(core_axis_name="c", subcore_axis_name="s"),
    scratch_types=[pltpu.VMEM((_CL, _D), jnp.float32),
                   pltpu.VMEM((_CL, _D), jnp.float32),
                   pltpu.VMEM((_D,), jnp.float32),
                   pltpu.VMEM((_B,), jnp.float32),
                   pltpu.VMEM((_CL,), jnp.float32),
                   pltpu.VMEM((1, _D), jnp.float32),
                   pltpu.VMEM((1, 16), jnp.float32),
                   pltpu.SemaphoreType.DMA,
                   pltpu.SemaphoreType.DMA],
)(_sc_attn_body)


def _tc_attn_body(ne_ref, v_ref, c_ref, agg_ref):
    b = pl.program_id(0)
    ne = ne_ref[...]
    scores = lax.dot_general(v_ref[...], ne, (((1,), (1,)), ((), ())),
                             preferred_element_type=jnp.float32)
    c_blk = c_ref[pl.ds(pl.multiple_of(b * 8, 8), _R2), :]
    aggs = []
    for r in range(_R2):
        s = scores[:, r * _L:(r + 1) * _L] + c_blk[r:r + 1, :]
        s = jnp.where(s >= 0.0, s, 0.2 * s)
        m = jnp.max(s, axis=1, keepdims=True)
        e = jnp.exp(s - m)
        alpha = e / jnp.sum(e, axis=1, keepdims=True)
        aggs.append(jnp.dot(alpha, ne[r * _L:(r + 1) * _L, :],
                            preferred_element_type=jnp.float32))
    agg_ref[0] = jnp.concatenate(aggs, axis=0)


def _tc_attn_call(ne_flat, v2, cslots):
    const = lambda *_: (0, 0)
    out3 = pl.pallas_call(
        _tc_attn_body,
        grid=(_TC_STEPS,),
        in_specs=[pl.BlockSpec((_R2 * _L, _D), lambda b: (b + _SC_ROWS // _R2, 0)),
                  pl.BlockSpec((1, _D), const),
                  pl.BlockSpec((_TC_STEPS * 8, 1), const)],
        out_specs=pl.BlockSpec((1, _R2, _D), lambda b: (b, 0, 0)),
        out_shape=jax.ShapeDtypeStruct((_TC_STEPS, _R2, _D), jnp.float32),
    )(ne_flat, v2, cslots)
    return out3.reshape(_TC_ROWS, _D)


def _mlp_body(state_ref, aggtc_ref, aggp_ref, stats_ref,
              w1_ref, b1_ref, w2_ref, b2_ref, w3_ref, b3_ref, out_ref):
    rows = []
    for r in range(_SC_ROWS):
        sl = aggp_ref[pl.ds(r * _WPR, _WPR), :]
        st = stats_ref[pl.ds(r * _WPR, _WPR), :]
        m4 = st[:, 0:1]
        s4 = st[:, 1:2]
        m = jnp.max(m4, axis=0, keepdims=True)
        scale = jnp.exp(m4 - m)
        num = jnp.sum(scale * sl, axis=0, keepdims=True)
        den = jnp.sum(scale * s4, axis=0, keepdims=True)
        rows.append(num / den)
    ag = jnp.concatenate(rows + [aggtc_ref[...]], axis=0)

    st_full = state_ref[...]
    h = jnp.dot(st_full, w1_ref[0:2 * _D, :], preferred_element_type=jnp.float32)
    h = h + jnp.dot(ag, w1_ref[2 * _D:3 * _D, :], preferred_element_type=jnp.float32)
    h = jax.nn.relu(h + b1_ref[...])
    h = jax.nn.relu(jnp.dot(h, w2_ref[...], preferred_element_type=jnp.float32) + b2_ref[...])
    logits = jnp.dot(h, w3_ref[...], preferred_element_type=jnp.float32) + b3_ref[...]
    z = logits - jnp.max(logits, axis=1, keepdims=True)
    ez = jnp.exp(z)
    out_ref[...] = ez / jnp.sum(ez, axis=1, keepdims=True)


def _mlp_call(state, agg_tc, aggp, stats, W1, b12, W2, b22, W3, b32):
    const = lambda *_: (0, 0)
    return pl.pallas_call(
        _mlp_body,
        grid=(1,),
        in_specs=[pl.BlockSpec((_B, 2 * _D), const),
                  pl.BlockSpec((_TC_ROWS, _D), const),
                  pl.BlockSpec((32, _D), const),
                  pl.BlockSpec((32, 16), const),
                  pl.BlockSpec((3 * _D, _H1), const),
                  pl.BlockSpec((1, _H1), const),
                  pl.BlockSpec((_H1, _H2), const),
                  pl.BlockSpec((1, _H2), const),
                  pl.BlockSpec((_H2, _ACT), const),
                  pl.BlockSpec((1, _ACT), const)],
        out_specs=pl.BlockSpec((_B, _ACT), const),
        out_shape=jax.ShapeDtypeStruct((_B, _ACT), jnp.float32),
    )(state, agg_tc, aggp, stats, W1, b12, W2, b22, W3, b32)


def kernel(state, now_embedding, neighbour_embeddings, Wa, ba, Ww, bw, W1, b1, W2, b2, W3, b3):
    ne_flat = neighbour_embeddings.reshape(_B * _L, _D)
    ba2 = ba.reshape(1, _A)
    bw2 = bw.reshape(1, 1)
    b12 = b1.reshape(1, _H1)
    b22 = b2.reshape(1, _H2)
    b32 = b3.reshape(1, _ACT)

    v2, c16, cslots = _prep_call(now_embedding, Wa, ba2, Ww, bw2)
    aggp, stats = _sc_attn_call(ne_flat, v2.reshape(_D), c16.reshape(_B))
    agg_tc = _tc_attn_call(ne_flat, v2, cslots)
    return _mlp_call(state, agg_tc, aggp, stats, W1, b12, W2, b22, W3, b32)


# speedup vs baseline: 3.5154x; 2.7101x over previous
import jax
import jax.numpy as jnp
from jax.experimental import pallas as pl
from jax.experimental.pallas import tpu as pltpu

_B, _L, _D, _A, _ACT = 16, 1024, 256, 128, 1024
_H1, _H2 = 512, 1024
_R = 8
_STEPS = _B // _R


def _attn_mlp_body(state_ref, now_ref, ne_ref, wa_ref, ba_ref, ww_ref, bw_ref,
                   w1_ref, b1_ref, w2_ref, b2_ref, w3_ref, b3_ref,
                   out_ref, agg_ref, v_scr, c_scr):
    b = pl.program_id(0)
    ne = ne_ref[...]

    @pl.when(b == 0)
    def _prep():
        wa = wa_ref[...]
        ww_top = ww_ref[0:_A, :]
        ww_bot = ww_ref[_A:2 * _A, :]
        v_scr[...] = jax.lax.dot_general(
            ww_bot, wa, (((0,), (1,)), ((), ())),
            preferred_element_type=jnp.float32)
        u = jnp.dot(wa, ww_top, preferred_element_type=jnp.float32)
        c_all = jnp.dot(now_ref[...], u, preferred_element_type=jnp.float32)
        c_all = c_all + jnp.dot(ba_ref[...], ww_top, preferred_element_type=jnp.float32)
        c_all = c_all + jnp.dot(ba_ref[...], ww_bot, preferred_element_type=jnp.float32)
        c_all = c_all + bw_ref[...]
        for s in range(_STEPS):
            c_scr[pl.ds(s * 8, _R), :] = c_all[s * _R:(s + 1) * _R, :]

    scores = jax.lax.dot_general(
        v_scr[...], ne, (((1,), (1,)), ((), ())),
        preferred_element_type=jnp.float32)
    c_blk = c_scr[pl.ds(pl.multiple_of(b * 8, 8), _R), :]

    aggs = []
    for r in range(_R):
        s = scores[:, r * _L:(r + 1) * _L] + c_blk[r:r + 1, :]
        s = jnp.where(s >= 0.0, s, 0.2 * s)
        m = jnp.max(s, axis=1, keepdims=True)
        e = jnp.exp(s - m)
        alpha = e / jnp.sum(e, axis=1, keepdims=True)
        aggs.append(jnp.dot(alpha, ne[r * _L:(r + 1) * _L, :],
                            preferred_element_type=jnp.float32))
    agg_ref[pl.ds(pl.multiple_of(b * 8, 8), _R), :] = jnp.concatenate(aggs, axis=0)

    @pl.when(b == _STEPS - 1)
    def _mlp():
        st = state_ref[...]
        ag = jnp.concatenate(
            [agg_ref[pl.ds(s * 8, _R), :] for s in range(_STEPS)], axis=0)
        h = jnp.dot(st, w1_ref[0:2 * _D, :], preferred_element_type=jnp.float32)
        h = h + jnp.dot(ag, w1_ref[2 * _D:3 * _D, :], preferred_element_type=jnp.float32)
        h = jax.nn.relu(h + b1_ref[...])
        h = jax.nn.relu(jnp.dot(h, w2_ref[...], preferred_element_type=jnp.float32) + b2_ref[...])
        logits = jnp.dot(h, w3_ref[...], preferred_element_type=jnp.float32) + b3_ref[...]
        z = logits - jnp.max(logits, axis=1, keepdims=True)
        ez = jnp.exp(z)
        out_ref[...] = ez / jnp.sum(ez, axis=1, keepdims=True)


def kernel(state, now_embedding, neighbour_embeddings, Wa, ba, Ww, bw, W1, b1, W2, b2, W3, b3):
    ne_flat = neighbour_embeddings.reshape(_B * _L, _D)
    ba2 = ba.reshape(1, _A)
    bw2 = bw.reshape(1, 1)
    b12 = b1.reshape(1, _H1)
    b22 = b2.reshape(1, _H2)
    b32 = b3.reshape(1, _ACT)

    const = lambda *_: (0, 0)
    in_specs = [
            pl.BlockSpec((_B, 2 * _D), const),
            pl.BlockSpec((_B, _D), const),
            pl.BlockSpec((_R * _L, _D), lambda b: (b, 0)),
            pl.BlockSpec((_D, _A), const),
            pl.BlockSpec((1, _A), const),
            pl.BlockSpec((2 * _A, 1), const),
            pl.BlockSpec((1, 1), const),
            pl.BlockSpec((3 * _D, _H1), const),
            pl.BlockSpec((1, _H1), const),
            pl.BlockSpec((_H1, _H2), const),
            pl.BlockSpec((1, _H2), const),
            pl.BlockSpec((_H2, _ACT), const),
            pl.BlockSpec((1, _ACT), const),
    ]
    return pl.pallas_call(
        _attn_mlp_body,
        grid=(_STEPS,),
        in_specs=in_specs,
        out_specs=pl.BlockSpec((_B, _ACT), const),
        out_shape=jax.ShapeDtypeStruct((_B, _ACT), jnp.float32),
        scratch_shapes=[pltpu.VMEM((_STEPS * 8, _D), jnp.float32),
                        pltpu.VMEM((1, _D), jnp.float32),
                        pltpu.VMEM((_STEPS * 8, 1), jnp.float32)],
    )(state, now_embedding, ne_flat, Wa, ba2, Ww, bw2,
      W1, b12, W2, b22, W3, b32)
